# baseline (speedup 1.0000x reference)
import functools

import jax
import jax.numpy as jnp
from jax import lax
from jax.experimental import pallas as pl
from jax.experimental.pallas import tpu as pltpu
from jax.experimental.pallas import tpu_sc as plsc

B = 16384
F = 26
V = 100000
D = 32
N_NUM = 13
C_OUT = F * D + N_NUM

NC = 2
NS = 16
L = 16
NW = NC * NS
PLANES = F * D
PPW = PLANES // NW
HB = B // 2
HQ = B // 4
NVEC = HQ // L

_mesh = plsc.VectorSubcoreMesh(core_axis_name="c", subcore_axis_name="s")


@functools.partial(
    pl.kernel,
    mesh=_mesh,
    compiler_params=pltpu.CompilerParams(
        use_tc_tiling_on_sc=True,
        needs_layout_passes=False,
    ),
    out_type=jax.ShapeDtypeStruct((C_OUT, B), jnp.float32),
    scratch_types=[
        pltpu.VMEM((V,), jnp.float32),
        pltpu.VMEM((B,), jnp.int32),
        pltpu.VMEM((2, HQ), jnp.float32),
        pltpu.SemaphoreType.DMA,
    ],
)
def _emb_planes(table, xcat_t, xnum_t, out, plane_v, idx_v, val_v, sem_o):
    wid = lax.axis_index("s") * NC + lax.axis_index("c")

    @pl.when(wid < N_NUM)
    def _():
        for h in range(4):
            pltpu.sync_copy(xnum_t.at[wid, pl.ds(h * HQ, HQ)], val_v.at[0])
            pltpu.sync_copy(val_v.at[0], out.at[PLANES + wid, pl.ds(h * HQ, HQ)])

    def plane_body(i, carry):
        p = wid * PPW + i
        f = p // D
        d = p % D

        @pl.when((d == 0) | (i == 0))
        def _():
            pltpu.sync_copy(xcat_t.at[f], idx_v)

        pltpu.sync_copy(table.at[f, d], plane_v)
        copies = []
        for q in range(4):
            slot = q % 2
            if q >= 2:
                copies[q - 2].wait()

            @plsc.parallel_loop(0, NVEC, unroll=8)
            def _(j):
                val_v[slot, pl.ds(j * L, L)] = plsc.load_gather(
                    plane_v, [idx_v[pl.ds(q * HQ + j * L, L)]]
                )

            copies.append(
                pltpu.async_copy(val_v.at[slot], out.at[p, pl.ds(q * HQ, HQ)], sem_o)
            )
        copies[2].wait()
        copies[3].wait()
        return carry

    lax.fori_loop(0, PPW, plane_body, 0)


def kernel(X_num, X_cat, tables):
    t3 = jnp.transpose(tables, (0, 2, 1))
    xcat_t = X_cat.astype(jnp.int32).T
    xnum_t = X_num.T
    cols = _emb_planes(t3, xcat_t, xnum_t)
    return cols.T


# speedup vs baseline: 1.0165x; 1.0165x over previous
import functools

import jax
import jax.numpy as jnp
from jax import lax
from jax.experimental import pallas as pl
from jax.experimental.pallas import tpu as pltpu
from jax.experimental.pallas import tpu_sc as plsc

B = 16384
F = 26
V = 100000
D = 32
N_NUM = 13
C_OUT = F * D + N_NUM

NC = 2
NS = 16
L = 16
NW = NC * NS
PLANES = F * D
PPW = PLANES // NW
HB = B // 2
NVEC = HB // L

_mesh = plsc.VectorSubcoreMesh(core_axis_name="c", subcore_axis_name="s")


@functools.partial(
    pl.kernel,
    mesh=_mesh,
    compiler_params=pltpu.CompilerParams(
        use_tc_tiling_on_sc=True,
        needs_layout_passes=False,
    ),
    out_type=jax.ShapeDtypeStruct((C_OUT, B), jnp.float32),
    scratch_types=[
        pltpu.VMEM((V,), jnp.float32),
        pltpu.VMEM((B,), jnp.int32),
        pltpu.VMEM((HB,), jnp.float32),
    ],
)
def _emb_planes(table, xcat_t, xnum_t, out, plane_v, idx_v, val_v):
    wid = lax.axis_index("s") * NC + lax.axis_index("c")

    @pl.when(wid < N_NUM)
    def _():
        for h in range(2):
            pltpu.sync_copy(xnum_t.at[wid, pl.ds(h * HB, HB)], val_v)
            pltpu.sync_copy(val_v, out.at[PLANES + wid, pl.ds(h * HB, HB)])

    def plane_body(i, carry):
        p = wid * PPW + i
        f = p // D
        d = p % D

        @pl.when((d == 0) | (i == 0))
        def _():
            pltpu.sync_copy(xcat_t.at[f], idx_v)

        pltpu.sync_copy(table.at[f, d], plane_v)
        for h in range(2):

            @plsc.parallel_loop(0, NVEC, unroll=16)
            def _(j):
                val_v[pl.ds(j * L, L)] = plsc.load_gather(
                    plane_v, [idx_v[pl.ds(h * HB + j * L, L)]]
                )

            pltpu.sync_copy(val_v, out.at[p, pl.ds(h * HB, HB)])
        return carry

    lax.fori_loop(0, PPW, plane_body, 0)


def kernel(X_num, X_cat, tables):
    t3 = jnp.transpose(tables, (0, 2, 1))
    xcat_t = X_cat.astype(jnp.int32).T
    xnum_t = X_num.T
    cols = _emb_planes(t3, xcat_t, xnum_t)
    return cols.T
